# Initial kernel scaffold; baseline (speedup 1.0000x reference)
import jax
import jax.numpy as jnp
from jax.experimental import pallas as pl


def kernel(x, wires_p0, chans_p0, wires_p1, chans_p1, wires_p2, chans_p2, gi01, gi12, gi20, rc01, rc12, rc20):
    raise NotImplementedError("write your pallas kernel here")


# speedup vs baseline: 1.4198x; 1.4198x over previous
import functools

import jax
import jax.numpy as jnp
from jax import lax
from jax.experimental import pallas as pl
from jax.experimental.pallas import tpu as pltpu
from jax.experimental.pallas import tpu_sc as plsc

B, NF, NCH, NT = 2, 2, 1536, 128
NW0, NW1, NW2 = 476, 476, 584
NX = 10000
NS = B * NT
ROW = 3 * NX * 10

F0 = 0
F1 = F0 + 2 * NW0
F2 = F1 + 2 * NW1
NFEAT = F2 + 2 * NW2
W0 = NFEAT
C0 = W0 + NW0
W1 = C0 + NW0
C1 = W1 + NW1
W2 = C1 + NW1
C2 = W2 + NW2
RC01 = C2 + NW2
RC12 = RC01 + 2 * NX
RC20 = RC12 + 2 * NX
VLEN = RC20 + 2 * NX
NCONST = VLEN - NFEAT

XSRC = NF * NCH
ZIDX = XSRC
XBUF = XSRC + 16

NTILES = 32
CH = 9376
CH_LAST = ROW - (NTILES - 1) * CH
GPAD = NTILES * CH


def _sc_body(xT, cvec, invi, gidx, out, V, gv, ob, xbt, invv):
    cid = lax.axis_index("c")
    sid = lax.axis_index("s")
    wid = sid * 2 + cid
    base = wid * CH

    pltpu.sync_copy(cvec, V.at[pl.ds(NFEAT, NCONST)])
    pltpu.sync_copy(invi, invv)
    pltpu.sync_copy(gidx.at[pl.ds(base, CH)], gv)
    xbt[pl.ds(ZIDX, 16)] = jnp.zeros((16,), jnp.float32)

    def slice_body(s, carry):
        xoff = pl.multiple_of(s * XSRC, 8)
        pltpu.sync_copy(xT.at[pl.ds(xoff, XSRC)], xbt.at[pl.ds(0, XSRC)])

        def feat_body(j, c):
            o = pl.multiple_of(j * 16, 16)
            idx = invv[pl.ds(o, 16)]
            V[pl.ds(o, 16)] = plsc.load_gather(xbt, [idx])
            return c

        lax.fori_loop(0, NFEAT // 16, feat_body, 0)

        def gather_body(i, c):
            o = pl.multiple_of(i * 16, 16)
            idx = gv[pl.ds(o, 16)]
            ob[pl.ds(o, 16)] = plsc.load_gather(V, [idx])
            return c

        lax.fori_loop(0, CH // 16, gather_body, 0)

        ooff = pl.multiple_of(s * ROW + base, 8)

        @pl.when(wid < NTILES - 1)
        def _():
            pltpu.sync_copy(ob, out.at[pl.ds(ooff, CH)])

        @pl.when(wid == NTILES - 1)
        def _():
            pltpu.sync_copy(ob.at[pl.ds(0, CH_LAST)],
                            out.at[pl.ds(ooff, CH_LAST)])

        return carry

    lax.fori_loop(0, NS, slice_body, 0)


def _routing(w_ids, c_ids, nw):
    jmax = jnp.full((nw,), -1, jnp.int32).at[w_ids].max(
        jnp.arange(nw, dtype=jnp.int32))
    ch = jnp.where(jmax >= 0, c_ids[jnp.clip(jmax, 0, nw - 1)], -1)
    i0 = jnp.where(ch >= 0, ch, ZIDX)
    i1 = jnp.where(ch >= 0, ch + NCH, ZIDX)
    return jnp.stack([i0, i1], axis=1).reshape(-1).astype(jnp.int32)


def _cross_idx(gi, fa, wa, ca, fb, wb, cb, rc):
    k = jnp.arange(NX, dtype=jnp.int32)
    a = fa + gi[:, 0] * 2
    b = fb + gi[:, 1] * 2
    cols = jnp.stack([
        a, a + 1, wa + gi[:, 0], ca + gi[:, 0],
        b, b + 1, wb + gi[:, 1], cb + gi[:, 1],
        rc + 2 * k, rc + 2 * k + 1,
    ], axis=1)
    return cols.reshape(-1)


def kernel(x, wires_p0, chans_p0, wires_p1, chans_p1, wires_p2, chans_p2,
           gi01, gi12, gi20, rc01, rc12, rc20):
    f32 = jnp.float32
    xT = jnp.transpose(x, (0, 3, 1, 2)).reshape(NS * XSRC)

    invidx = jnp.concatenate([
        _routing(wires_p0, chans_p0, NW0),
        _routing(wires_p1, chans_p1, NW1),
        _routing(wires_p2, chans_p2, NW2),
    ])

    cvec = jnp.concatenate([
        wires_p0.astype(f32), chans_p0.astype(f32),
        wires_p1.astype(f32), chans_p1.astype(f32),
        wires_p2.astype(f32), chans_p2.astype(f32),
        rc01.reshape(-1), rc12.reshape(-1), rc20.reshape(-1),
    ])

    g = jnp.concatenate([
        _cross_idx(gi01, F0, W0, C0, F1, W1, C1, RC01),
        _cross_idx(gi12, F1, W1, C1, F2, W2, C2, RC12),
        _cross_idx(gi20, F2, W2, C2, F0, W0, C0, RC20),
        jnp.zeros((GPAD - ROW,), jnp.int32),
    ]).astype(jnp.int32)

    mesh = plsc.VectorSubcoreMesh(core_axis_name="c", subcore_axis_name="s",
                                  num_cores=2, num_subcores=16)
    run = pl.kernel(
        _sc_body,
        out_type=jax.ShapeDtypeStruct((NS * ROW,), f32),
        mesh=mesh,
        compiler_params=pltpu.CompilerParams(needs_layout_passes=False),
        scratch_types=[
            pltpu.VMEM((VLEN,), f32),
            pltpu.VMEM((CH,), jnp.int32),
            pltpu.VMEM((CH,), f32),
            pltpu.VMEM((XBUF,), f32),
            pltpu.VMEM((NFEAT,), jnp.int32),
        ],
    )
    out = run(xT, cvec, invidx, g)
    return out.reshape(B, NT, 3 * NX, 10)


# speedup vs baseline: 1.6905x; 1.1907x over previous
import functools

import jax
import jax.numpy as jnp
from jax import lax
from jax.experimental import pallas as pl
from jax.experimental.pallas import tpu as pltpu
from jax.experimental.pallas import tpu_sc as plsc

B, NF, NCH, NT = 2, 2, 1536, 128
NW0, NW1, NW2 = 476, 476, 584
NX = 10000
NS = B * NT
ROW = 3 * NX * 10

F0 = 0
F1 = F0 + 2 * NW0
F2 = F1 + 2 * NW1
NFEAT = F2 + 2 * NW2
W0 = NFEAT
C0 = W0 + NW0
W1 = C0 + NW0
C1 = W1 + NW1
W2 = C1 + NW1
C2 = W2 + NW2
RC01 = C2 + NW2
RC12 = RC01 + 2 * NX
RC20 = RC12 + 2 * NX
VLEN = RC20 + 2 * NX
NCONST = VLEN - NFEAT

XSRC = NF * NCH
ZIDX = XSRC
XBUF = XSRC + 16

NTILES = 32
CH = 9376
CH_LAST = ROW - (NTILES - 1) * CH
GPAD = NTILES * CH


CH_MAIN = 9344


def _sc_body(xT, cvec, invi, gidx, out,
             V, gv, obA, obB, xbtA, xbtB, invv, sxA, sxB, soA, soB):
    cid = lax.axis_index("c")
    sid = lax.axis_index("s")
    wid = sid * 2 + cid
    base = wid * CH

    pltpu.sync_copy(cvec, V.at[pl.ds(NFEAT, NCONST)])
    pltpu.sync_copy(invi, invv)
    pltpu.sync_copy(gidx.at[pl.ds(base, CH)], gv)
    zeros = jnp.zeros((16,), jnp.float32)
    xbtA[pl.ds(ZIDX, 16)] = zeros
    xbtB[pl.ds(ZIDX, 16)] = zeros

    def start_x(s, xbt, sx):
        xoff = pl.multiple_of(s * XSRC, 8)
        pltpu.async_copy(xT.at[pl.ds(xoff, XSRC)], xbt.at[pl.ds(0, XSRC)], sx)

    def wait_x(xbt, sx):
        pltpu.make_async_copy(xT.at[pl.ds(0, XSRC)],
                              xbt.at[pl.ds(0, XSRC)], sx).wait()

    def out_copy(s, ob, so):
        ooff = pl.multiple_of(s * ROW + base, 8)
        full = pltpu.make_async_copy(ob, out.at[pl.ds(ooff, CH)], so)
        last = pltpu.make_async_copy(ob.at[pl.ds(0, CH_LAST)],
                                     out.at[pl.ds(ooff, CH_LAST)], so)
        return full, last

    def start_out(s, ob, so):
        full, last = out_copy(s, ob, so)
        pl.when(wid < NTILES - 1)(full.start)
        pl.when(wid == NTILES - 1)(last.start)

    def wait_out(s, ob, so):
        full, last = out_copy(s, ob, so)
        pl.when(wid < NTILES - 1)(full.wait)
        pl.when(wid == NTILES - 1)(last.wait)

    def feat_build(xbt):
        @plsc.parallel_loop(0, NFEAT, 16, unroll=8)
        def _(o):
            idx = invv[pl.ds(o, 16)]
            V[pl.ds(o, 16)] = plsc.load_gather(xbt, [idx])

    def gather_to(ob):
        @plsc.parallel_loop(0, CH_MAIN, 16, unroll=8)
        def _(o):
            idx = gv[pl.ds(o, 16)]
            ob[pl.ds(o, 16)] = plsc.load_gather(V, [idx])
        for o in range(CH_MAIN, CH, 16):
            idx = gv[pl.ds(o, 16)]
            ob[pl.ds(o, 16)] = plsc.load_gather(V, [idx])

    start_x(0, xbtA, sxA)

    def pair_body(g, carry):
        sA = 2 * g
        sB = sA + 1
        wait_x(xbtA, sxA)
        start_x(sB, xbtB, sxB)
        feat_build(xbtA)
        pl.when(g > 0)(lambda: wait_out(sA - 2, obA, soA))
        gather_to(obA)
        start_out(sA, obA, soA)
        wait_x(xbtB, sxB)
        pl.when(g < NS // 2 - 1)(lambda: start_x(sB + 1, xbtA, sxA))
        feat_build(xbtB)
        pl.when(g > 0)(lambda: wait_out(sB - 2, obB, soB))
        gather_to(obB)
        start_out(sB, obB, soB)
        return carry

    lax.fori_loop(0, NS // 2, pair_body, 0)
    wait_out(NS - 2, obA, soA)
    wait_out(NS - 1, obB, soB)


def _routing(w_ids, c_ids, nw):
    jmax = jnp.full((nw,), -1, jnp.int32).at[w_ids].max(
        jnp.arange(nw, dtype=jnp.int32))
    ch = jnp.where(jmax >= 0, c_ids[jnp.clip(jmax, 0, nw - 1)], -1)
    i0 = jnp.where(ch >= 0, ch, ZIDX)
    i1 = jnp.where(ch >= 0, ch + NCH, ZIDX)
    return jnp.stack([i0, i1], axis=1).reshape(-1).astype(jnp.int32)


def _cross_idx(gi, fa, wa, ca, fb, wb, cb, rc):
    k = jnp.arange(NX, dtype=jnp.int32)
    a = fa + gi[:, 0] * 2
    b = fb + gi[:, 1] * 2
    cols = jnp.stack([
        a, a + 1, wa + gi[:, 0], ca + gi[:, 0],
        b, b + 1, wb + gi[:, 1], cb + gi[:, 1],
        rc + 2 * k, rc + 2 * k + 1,
    ], axis=1)
    return cols.reshape(-1)


def kernel(x, wires_p0, chans_p0, wires_p1, chans_p1, wires_p2, chans_p2,
           gi01, gi12, gi20, rc01, rc12, rc20):
    f32 = jnp.float32
    xT = jnp.transpose(x, (0, 3, 1, 2)).reshape(NS * XSRC)

    invidx = jnp.concatenate([
        _routing(wires_p0, chans_p0, NW0),
        _routing(wires_p1, chans_p1, NW1),
        _routing(wires_p2, chans_p2, NW2),
    ])

    cvec = jnp.concatenate([
        wires_p0.astype(f32), chans_p0.astype(f32),
        wires_p1.astype(f32), chans_p1.astype(f32),
        wires_p2.astype(f32), chans_p2.astype(f32),
        rc01.reshape(-1), rc12.reshape(-1), rc20.reshape(-1),
    ])

    g = jnp.concatenate([
        _cross_idx(gi01, F0, W0, C0, F1, W1, C1, RC01),
        _cross_idx(gi12, F1, W1, C1, F2, W2, C2, RC12),
        _cross_idx(gi20, F2, W2, C2, F0, W0, C0, RC20),
        jnp.zeros((GPAD - ROW,), jnp.int32),
    ]).astype(jnp.int32)

    mesh = plsc.VectorSubcoreMesh(core_axis_name="c", subcore_axis_name="s",
                                  num_cores=2, num_subcores=16)
    run = pl.kernel(
        _sc_body,
        out_type=jax.ShapeDtypeStruct((NS * ROW,), f32),
        mesh=mesh,
        compiler_params=pltpu.CompilerParams(needs_layout_passes=False),
        scratch_types=[
            pltpu.VMEM((VLEN,), f32),
            pltpu.VMEM((CH,), jnp.int32),
            pltpu.VMEM((CH,), f32),
            pltpu.VMEM((CH,), f32),
            pltpu.VMEM((XBUF,), f32),
            pltpu.VMEM((XBUF,), f32),
            pltpu.VMEM((NFEAT,), jnp.int32),
            pltpu.SemaphoreType.DMA,
            pltpu.SemaphoreType.DMA,
            pltpu.SemaphoreType.DMA,
            pltpu.SemaphoreType.DMA,
        ],
    )
    out = run(xT, cvec, invidx, g)
    return out.reshape(B, NT, 3 * NX, 10)


# speedup vs baseline: 2.7815x; 1.6454x over previous
import jax
import jax.numpy as jnp
from jax import lax
from jax.experimental import pallas as pl
from jax.experimental.pallas import tpu as pltpu
from jax.experimental.pallas import tpu_sc as plsc

B, NF, NCH, NT = 2, 2, 1536, 128
NW0, NW1, NW2 = 476, 476, 584
NX = 10000
NK3 = 3 * NX
NROWS = B * NF * NCH
ZROW = NROWS
XROWS = NROWS + 8

NWSUM = NW0 + NW1 + NW2
P0, P1, P2 = 0, NW0, NW0 + NW1

NK = 80
NCHUNK = NK3 // NK
NTILES = 32
JMAX = (NCHUNK + NTILES - 1) // NTILES

def _sc_body(xext, giA, giB, rc0, rc1, rowc, wiresf, chansf, out,
             rowc_v, wires_v, chans_v, giav, gibv, rc0v, rc1v,
             Lg, stage, bb, vs16, semL, semG, semOF, semOB):
    cid = lax.axis_index("c")
    sid = lax.axis_index("s")
    wid = sid * 2 + cid

    pltpu.sync_copy(rowc, rowc_v)
    pltpu.sync_copy(wiresf, wires_v)
    pltpu.sync_copy(chansf, chans_v)

    def chunk_body(i):
        q0 = pl.multiple_of(i * NK, 8)
        g = lax.div(i, jnp.int32(NCHUNK // 3))
        pa = g * NW0
        pb = jnp.where(g < 2, pa + NW0, 0)

        cp1 = pltpu.make_async_copy(giA.at[pl.ds(q0, NK)], giav, semL)
        cp2 = pltpu.make_async_copy(giB.at[pl.ds(q0, NK)], gibv, semL)
        cp3 = pltpu.make_async_copy(rc0.at[pl.ds(q0, NK)], rc0v, semL)
        cp4 = pltpu.make_async_copy(rc1.at[pl.ds(q0, NK)], rc1v, semL)
        for c in (cp1, cp2, cp3, cp4):
            c.start()
        for c in (cp1, cp2, cp3, cp4):
            c.wait()

        def v16_body(v, carry):
            o = pl.multiple_of(v * 16, 16)
            gA = giav[pl.ds(o, 16)] + pa
            gB = gibv[pl.ds(o, 16)] + pb
            cA = plsc.load_gather(rowc_v, [gA])
            cB = plsc.load_gather(rowc_v, [gB])
            okA = cA >= 0
            okB = cB >= 0
            zr = jnp.full((16,), ZROW, jnp.int32)
            Lg[0, pl.ds(o, 16)] = jnp.where(okA, cA, zr)
            Lg[1, pl.ds(o, 16)] = jnp.where(okA, cA + NCH, zr)
            Lg[2, pl.ds(o, 16)] = jnp.where(okB, cB, zr)
            Lg[3, pl.ds(o, 16)] = jnp.where(okB, cB + NCH, zr)
            Lg[4, pl.ds(o, 16)] = jnp.where(okA, cA + 2 * NCH, zr)
            Lg[5, pl.ds(o, 16)] = jnp.where(okA, cA + 3 * NCH, zr)
            Lg[6, pl.ds(o, 16)] = jnp.where(okB, cB + 2 * NCH, zr)
            Lg[7, pl.ds(o, 16)] = jnp.where(okB, cB + 3 * NCH, zr)

            vals = [
                plsc.load_gather(wires_v, [gA]),
                plsc.load_gather(chans_v, [gA]),
                plsc.load_gather(wires_v, [gB]),
                plsc.load_gather(chans_v, [gB]),
                rc0v[pl.ds(o, 16)],
                rc1v[pl.ds(o, 16)],
            ]
            rtzero = jnp.minimum(giav[pl.ds(0, 16)], 0)
            for sec, vec in enumerate(vals):
                vs16[:] = vec
                for kk in range(16):
                    sp = plsc.load_gather(vs16, [rtzero + kk])
                    for L in range(8):
                        bb[sec, o + kk, pl.ds(L * 16, 16)] = sp
            return carry

        lax.fori_loop(0, NK // 16, v16_body, 0)

        def gathers(b):
            cps = [pltpu.make_async_copy(xext.at[Lg.at[4 * b + r]],
                                         stage.at[r], semG)
                   for r in range(4)]
            return cps

        def outs_feat(b, so):
            cs = (0, 1, 4, 5)
            cps = [pltpu.make_async_copy(
                stage.at[r], out.at[b, cs[r], pl.ds(q0, NK), :], so)
                for r in range(4)]
            return cps

        def outs_bb(b, so):
            cs = (2, 3, 6, 7, 8, 9)
            cps = [pltpu.make_async_copy(
                bb.at[sec], out.at[b, cs[sec], pl.ds(q0, NK), :], so)
                for sec in range(6)]
            return cps

        g0 = gathers(0)
        for c in g0:
            c.start()
        bb0 = outs_bb(0, semOB) + outs_bb(1, semOB)
        for c in bb0:
            c.start()
        for c in g0:
            c.wait()
        f0 = outs_feat(0, semOF)
        for c in f0:
            c.start()
        for c in f0:
            c.wait()
        g1 = gathers(1)
        for c in g1:
            c.start()
        for c in g1:
            c.wait()
        f1 = outs_feat(1, semOF)
        for c in f1:
            c.start()
        for c in f1:
            c.wait()
        for c in bb0:
            c.wait()

    def j_body(j, carry):
        i = wid + NTILES * j

        @pl.when(i < NCHUNK)
        def _():
            chunk_body(i)

        return carry

    lax.fori_loop(0, JMAX, j_body, 0)


def _routing(w_ids, c_ids, nw):
    jmax = jnp.full((nw,), -1, jnp.int32).at[w_ids].max(
        jnp.arange(nw, dtype=jnp.int32))
    return jnp.where(jmax >= 0, c_ids[jnp.clip(jmax, 0, nw - 1)],
                     -1).astype(jnp.int32)


def kernel(x, wires_p0, chans_p0, wires_p1, chans_p1, wires_p2, chans_p2,
           gi01, gi12, gi20, rc01, rc12, rc20):
    f32 = jnp.float32
    xext = jnp.concatenate(
        [x.reshape(NROWS, NT), jnp.zeros((XROWS - NROWS, NT), f32)])

    rowc = jnp.concatenate([
        _routing(wires_p0, chans_p0, NW0),
        _routing(wires_p1, chans_p1, NW1),
        _routing(wires_p2, chans_p2, NW2),
    ])
    wiresf = jnp.concatenate([wires_p0, wires_p1, wires_p2]).astype(f32)
    chansf = jnp.concatenate([chans_p0, chans_p1, chans_p2]).astype(f32)

    giA = jnp.concatenate([gi01[:, 0], gi12[:, 0], gi20[:, 0]])
    giB = jnp.concatenate([gi01[:, 1], gi12[:, 1], gi20[:, 1]])
    rc0 = jnp.concatenate([rc01[:, 0], rc12[:, 0], rc20[:, 0]])
    rc1 = jnp.concatenate([rc01[:, 1], rc12[:, 1], rc20[:, 1]])

    mesh = plsc.VectorSubcoreMesh(core_axis_name="c", subcore_axis_name="s",
                                  num_cores=2, num_subcores=16)
    run = pl.kernel(
        _sc_body,
        out_type=jax.ShapeDtypeStruct((B, 10, NK3, NT), f32),
        mesh=mesh,
        compiler_params=pltpu.CompilerParams(needs_layout_passes=False),
        scratch_types=[
            pltpu.VMEM((NWSUM,), jnp.int32),
            pltpu.VMEM((NWSUM,), f32),
            pltpu.VMEM((NWSUM,), f32),
            pltpu.VMEM((NK,), jnp.int32),
            pltpu.VMEM((NK,), jnp.int32),
            pltpu.VMEM((NK,), f32),
            pltpu.VMEM((NK,), f32),
            pltpu.VMEM((8, NK), jnp.int32),
            pltpu.VMEM((4, NK, NT), f32),
            pltpu.VMEM((6, NK, NT), f32),
            pltpu.VMEM((16,), f32),
            pltpu.SemaphoreType.DMA,
            pltpu.SemaphoreType.DMA,
            pltpu.SemaphoreType.DMA,
            pltpu.SemaphoreType.DMA,
        ],
    )
    out = run(xext, giA, giB, rc0, rc1, rowc, wiresf, chansf)
    return jnp.transpose(out, (0, 3, 2, 1))
